# baseline (speedup 1.0000x reference)
import jax
import jax.numpy as jnp
from jax import lax
from jax.experimental import pallas as pl
from jax.experimental.pallas import tpu as pltpu
from jax.experimental.pallas import tpu_sc as plsc

VOCAB = 1000000
EMBED_DIM = 64
NUM_CLASSES = 2
BATCH = 4096
HIST = 200

NC = 2
NS = 16
NW = NC * NS
D16 = EMBED_DIM // 16

TBLK = 512
NUNITS = VOCAB // TBLK
TAIL = VOCAB - NUNITS * TBLK
UNITS_PER_W = -(-NUNITS // NW)
SUB = 128

ROWS_PER_W = BATCH // NW
CH = (104, 96)
CHOFF = (0, 104)
CHMAX = 104
NCHUNKS = ROWS_PER_W * 2
NBUF = 4


def _transpose_body(tt_hbm, tail_hbm, out_hbm, in_v, out_v, tail_v, sem):
    cid = lax.axis_index("c")
    sid = lax.axis_index("s")
    wid = sid * NC + cid

    iotas = tuple(lax.broadcasted_iota(jnp.int32, (16,), 0) + 16 * k
                  for k in range(D16))

    def transpose_unit(off):
        pltpu.sync_copy(tt_hbm.at[:, pl.ds(off, TBLK)], in_v)
        for s in range(TBLK // SUB):
            @plsc.parallel_loop(0, SUB, step=1, unroll=8)
            def _(r):
                col = jnp.full((16,), s * SUB + r, jnp.int32)
                for k in range(D16):
                    vals = plsc.load_gather(in_v, [iotas[k], col])
                    out_v[r, pl.ds(16 * k, 16)] = vals
            pltpu.sync_copy(out_v,
                            out_hbm.at[pl.ds(off + s * SUB, SUB)])

    def unit_body(i, carry):
        del carry
        g = i * NW + wid

        @pl.when(g < NUNITS)
        def _():
            transpose_unit(g * TBLK)
        return 0

    lax.fori_loop(0, UNITS_PER_W, unit_body, 0)

    @pl.when(wid == 0)
    def _():
        pltpu.sync_copy(tail_hbm, tail_v)

        @plsc.parallel_loop(0, TAIL, step=1, unroll=2)
        def _(r):
            for k in range(D16):
                out_v[r, pl.ds(16 * k, 16)] = tail_v[r, pl.ds(16 * k, 16)]
        pltpu.sync_copy(out_v.at[pl.ds(0, TAIL)],
                        out_hbm.at[pl.ds(NUNITS * TBLK, TAIL)])


_transpose_call = pl.kernel(
    _transpose_body,
    out_type=jax.ShapeDtypeStruct((VOCAB, 128), jnp.float32),
    mesh=plsc.VectorSubcoreMesh(core_axis_name="c", subcore_axis_name="s"),
    scratch_types=[
        pltpu.VMEM((EMBED_DIM, TBLK), jnp.float32),
        pltpu.VMEM((SUB, 128), jnp.float32),
        pltpu.VMEM((TAIL, EMBED_DIM), jnp.float32),
        pltpu.SemaphoreType.DMA,
    ],
    compiler_params=pltpu.CompilerParams(needs_layout_passes=False),
)


def _sc_body(t128_hbm, x_hbm, wt_hbm, b_hbm, out_hbm,
             idx_v, bufs, wt_v, b_v, out_v, sems):
    cid = lax.axis_index("c")
    sid = lax.axis_index("s")
    wid = sid * NC + cid

    pltpu.sync_copy(x_hbm.at[pl.ds(wid * NCHUNKS, NCHUNKS)], idx_v)
    pltpu.sync_copy(wt_hbm, wt_v)
    pltpu.sync_copy(b_hbm, b_v)

    lane = lax.broadcasted_iota(jnp.int32, (16,), 0)
    zero = jnp.zeros((16,), jnp.float32)
    b_vec = b_v[...]
    wvecs = tuple(wt_v[c, pl.ds(k * 16, 16)]
                  for c in range(NUM_CLASSES) for k in range(D16))
    inv_l = jnp.float32(1.0 / HIST)

    def fire(slot, j):
        return pltpu.async_copy(t128_hbm.at[idx_v.at[j]],
                                bufs.at[slot], sems.at[slot])

    def wait(slot):
        pltpu.make_async_copy(t128_hbm.at[idx_v.at[0]],
                              bufs.at[slot], sems.at[slot]).wait()

    def reduce_buf(slot, half, acc):
        buf = bufs.at[slot]

        @plsc.parallel_loop(0, CH[half], step=1, unroll=4, carry=acc)
        def body(r, a):
            return tuple(a[k] + buf[r, pl.ds(k * 16, 16)] for k in range(D16))

        return body

    def finalize(row, acc):
        out_row = b_vec
        for c in range(NUM_CLASSES):
            s = jnp.float32(0.0)
            for k in range(D16):
                s = s + jnp.sum(acc[k] * wvecs[c * D16 + k])
            out_row = out_row + jnp.where(lane == c, s * inv_l, 0.0)
        out_v[row] = out_row

    for b in range(NBUF):
        fire(b, b)

    @pl.loop(0, NCHUNKS - NBUF, step=NBUF)
    def _(g):
        row = g >> 1
        for b in range(NBUF):
            wait(b)
            acc = (zero,) * D16 if b % 2 == 0 else acc2
            acc2 = reduce_buf(b, b % 2, acc)
            fire(b, g + b + NBUF)
            if b % 2 == 1:
                finalize(row + b // 2, acc2)

    for b in range(NBUF):
        j = NCHUNKS - NBUF + b
        wait(b)
        acc = (zero,) * D16 if b % 2 == 0 else acc2
        acc2 = reduce_buf(b, b % 2, acc)
        if b % 2 == 1:
            finalize(j // 2, acc2)

    pltpu.sync_copy(out_v, out_hbm.at[pl.ds(wid * ROWS_PER_W, ROWS_PER_W)])


_sc_call = pl.kernel(
    _sc_body,
    out_type=jax.ShapeDtypeStruct((BATCH, 16), jnp.float32),
    mesh=plsc.VectorSubcoreMesh(core_axis_name="c", subcore_axis_name="s"),
    scratch_types=[
        pltpu.VMEM((NCHUNKS, CHMAX), jnp.int32),
        pltpu.VMEM((NBUF, CHMAX, 128), jnp.float32),
        pltpu.VMEM((NUM_CLASSES, EMBED_DIM), jnp.float32),
        pltpu.VMEM((16,), jnp.float32),
        pltpu.VMEM((ROWS_PER_W, 16), jnp.float32),
        pltpu.SemaphoreType.DMA((NBUF,)),
    ],
    compiler_params=pltpu.CompilerParams(needs_layout_passes=False),
)


@jax.jit
def kernel(x, table, W, b):
    wt = W.T.astype(jnp.float32)
    b_pad = jnp.pad(b.astype(jnp.float32), (0, 16 - NUM_CLASSES))
    t128 = _transpose_call(table.T, table[NUNITS * TBLK:])
    xi = x.astype(jnp.int32)
    xa = xi[:, :CHMAX].reshape(BATCH, 1, CHMAX)
    xb = jnp.pad(xi[:, CHMAX:], ((0, 0), (0, CHMAX - CH[1])))
    x3 = jnp.concatenate([xa, xb.reshape(BATCH, 1, CHMAX)], axis=1)
    x3 = x3.reshape(BATCH * 2, CHMAX)
    out16 = _sc_call(t128, x3, wt, b_pad)
    return out16[:, :NUM_CLASSES]


# speedup vs baseline: 2.8283x; 2.8283x over previous
import jax
import jax.numpy as jnp
from jax import lax
from jax.experimental import pallas as pl
from jax.experimental.pallas import tpu as pltpu
from jax.experimental.pallas import tpu_sc as plsc

VOCAB = 1000000
EMBED_DIM = 64
NUM_CLASSES = 2
BATCH = 4096
HIST = 200

NC = 2
NS = 16
NW = NC * NS
ROWS_PER_W = BATCH // NW
CH = (104, 96)
CHOFF = (0, 104)
CHMAX = 104
CHUNKS_PER_ROW = 2
NCHUNKS = ROWS_PER_W * CHUNKS_PER_ROW
D16 = EMBED_DIM // 16
NBUF = 4


def _sc_body(table_hbm, x_hbm, wt_hbm, b_hbm, out_hbm,
             idx_v, bufs, wt_v, b_v, out_v, sems):
    cid = lax.axis_index("c")
    sid = lax.axis_index("s")
    wid = sid * NC + cid

    pltpu.sync_copy(x_hbm.at[pl.ds(wid * ROWS_PER_W, ROWS_PER_W)], idx_v)
    pltpu.sync_copy(wt_hbm, wt_v)
    pltpu.sync_copy(b_hbm, b_v)

    lane = lax.broadcasted_iota(jnp.int32, (16,), 0)
    zero = jnp.zeros((16,), jnp.float32)
    b_vec = b_v[...]
    wvecs = tuple(wt_v[c, pl.ds(k * 16, 16)]
                  for c in range(NUM_CLASSES) for k in range(D16))
    inv_l = jnp.float32(1.0 / HIST)

    def fire(slot, row, half):
        n = CH[half]
        idx = idx_v.at[row, pl.ds(CHOFF[half], n)]
        return pltpu.async_copy(table_hbm.at[idx],
                                bufs.at[slot, pl.ds(0, n)],
                                sems.at[slot])

    def wait(slot, half):
        n = CH[half]
        pltpu.make_async_copy(table_hbm.at[idx_v.at[0, pl.ds(0, n)]],
                              bufs.at[slot, pl.ds(0, n)], sems.at[slot]).wait()

    def reduce_buf(slot, half, acc):
        buf = bufs.at[slot]

        @plsc.parallel_loop(0, CH[half], step=1, unroll=4, carry=acc)
        def body(r, a):
            v0 = buf[r, pl.ds(0, 32)]
            v1 = buf[r, pl.ds(32, 32)]
            a0, b0 = plsc.unpack(v0, format=plsc.PackFormat.INTERLEAVED)
            a1, b1 = plsc.unpack(v1, format=plsc.PackFormat.INTERLEAVED)
            return (a[0] + a0, a[1] + b0, a[2] + a1, a[3] + b1)

        return body

    def finalize(row, acc):
        out_row = b_vec
        for c in range(NUM_CLASSES):
            s = jnp.float32(0.0)
            for k in range(D16):
                s = s + jnp.sum(acc[k] * wvecs[c * D16 + k])
            out_row = out_row + jnp.where(lane == c, s * inv_l, 0.0)
        out_v[row] = out_row

    for b in range(NBUF):
        fire(b, b // 2, b % 2)

    @pl.loop(0, NCHUNKS - NBUF, step=NBUF)
    def _(g):
        row = g >> 1
        for b in range(NBUF):
            wait(b, b % 2)
            acc = (zero,) * D16 if b % 2 == 0 else acc2
            acc2 = reduce_buf(b, b % 2, acc)
            nxt = g + b + NBUF
            fire(b, nxt >> 1, b % 2)
            if b % 2 == 1:
                finalize(row + b // 2, acc2)

    for b in range(NBUF):
        j = NCHUNKS - NBUF + b
        wait(b, b % 2)
        acc = (zero,) * D16 if b % 2 == 0 else acc2
        acc2 = reduce_buf(b, b % 2, acc)
        if b % 2 == 1:
            finalize(j // 2, acc2)

    pltpu.sync_copy(out_v, out_hbm.at[pl.ds(wid * ROWS_PER_W, ROWS_PER_W)])


_sc_call = pl.kernel(
    _sc_body,
    out_type=jax.ShapeDtypeStruct((BATCH, 16), jnp.float32),
    mesh=plsc.VectorSubcoreMesh(core_axis_name="c", subcore_axis_name="s"),
    scratch_types=[
        pltpu.VMEM((ROWS_PER_W, HIST), jnp.int32),
        pltpu.VMEM((NBUF, CHMAX, EMBED_DIM), jnp.bfloat16),
        pltpu.VMEM((NUM_CLASSES, EMBED_DIM), jnp.float32),
        pltpu.VMEM((16,), jnp.float32),
        pltpu.VMEM((ROWS_PER_W, 16), jnp.float32),
        pltpu.SemaphoreType.DMA((NBUF,)),
    ],
    compiler_params=pltpu.CompilerParams(
        needs_layout_passes=False, use_tc_tiling_on_sc=False),
)


@jax.jit
def kernel(x, table, W, b):
    wt = W.T.astype(jnp.float32)
    wt2 = jnp.concatenate([wt[:, 0:32:2], wt[:, 1:32:2],
                           wt[:, 32:64:2], wt[:, 33:64:2]], axis=1)
    b_pad = jnp.pad(b.astype(jnp.float32), (0, 16 - NUM_CLASSES))
    tb = table.astype(jnp.bfloat16)
    out16 = _sc_call(tb, x.astype(jnp.int32), wt2, b_pad)
    return out16[:, :NUM_CLASSES]


# speedup vs baseline: 3.6101x; 1.2765x over previous
import jax
import jax.numpy as jnp
from jax import lax
from jax.experimental import pallas as pl
from jax.experimental.pallas import tpu as pltpu
from jax.experimental.pallas import tpu_sc as plsc

VOCAB = 1000000
EMBED_DIM = 64
NUM_CLASSES = 2
BATCH = 4096
HIST = 200

NC = 2
NS = 16
NW = NC * NS
ROWS_PER_W = BATCH // NW
CH = (104, 96)
CHOFF = (0, 104)
CHMAX = 104
CHUNKS_PER_ROW = 2
NCHUNKS = ROWS_PER_W * CHUNKS_PER_ROW
D16 = EMBED_DIM // 16
NBUF = 4


def _sc_body(table_hbm, x_hbm, wt_hbm, b_hbm, out_hbm,
             idx_v, bufs, wt_v, b_v, out_v, sems):
    cid = lax.axis_index("c")
    sid = lax.axis_index("s")
    wid = sid * NC + cid

    pltpu.sync_copy(x_hbm.at[pl.ds(wid * ROWS_PER_W, ROWS_PER_W)], idx_v)
    pltpu.sync_copy(wt_hbm, wt_v)
    pltpu.sync_copy(b_hbm, b_v)

    lane = lax.broadcasted_iota(jnp.int32, (16,), 0)
    zero = jnp.zeros((16,), jnp.float32)
    b_vec = b_v[...]
    wvecs = tuple(wt_v[c, pl.ds(k * 16, 16)]
                  for c in range(NUM_CLASSES) for k in range(D16))
    inv_l = jnp.float32(1.0 / HIST)

    def fire(slot, row, half):
        n = CH[half]
        idx = idx_v.at[row, pl.ds(CHOFF[half], n)]
        return pltpu.async_copy(table_hbm.at[idx],
                                bufs.at[slot, pl.ds(0, n)],
                                sems.at[slot])

    def wait(slot, half):
        n = CH[half]
        pltpu.make_async_copy(table_hbm.at[idx_v.at[0, pl.ds(0, n)]],
                              bufs.at[slot, pl.ds(0, n)], sems.at[slot]).wait()

    def reduce_buf(slot, half, acc):
        buf = bufs.at[slot]

        @plsc.parallel_loop(0, CH[half], step=1, unroll=4, carry=acc)
        def body(r, a):
            return tuple(a[k] + buf[r, pl.ds(k * 16, 16)] for k in range(D16))

        return body

    def finalize(row, acc):
        out_row = b_vec
        for c in range(NUM_CLASSES):
            s = jnp.float32(0.0)
            for k in range(D16):
                s = s + jnp.sum(acc[k] * wvecs[c * D16 + k])
            out_row = out_row + jnp.where(lane == c, s * inv_l, 0.0)
        out_v[row] = out_row

    for b in range(NBUF):
        fire(b, b // 2, b % 2)

    @pl.loop(0, NCHUNKS - NBUF, step=NBUF)
    def _(g):
        row = g >> 1
        for b in range(NBUF):
            wait(b, b % 2)
            acc = (zero,) * D16 if b % 2 == 0 else acc2
            acc2 = reduce_buf(b, b % 2, acc)
            nxt = g + b + NBUF
            fire(b, nxt >> 1, b % 2)
            if b % 2 == 1:
                finalize(row + b // 2, acc2)

    for b in range(NBUF):
        j = NCHUNKS - NBUF + b
        wait(b, b % 2)
        acc = (zero,) * D16 if b % 2 == 0 else acc2
        acc2 = reduce_buf(b, b % 2, acc)
        if b % 2 == 1:
            finalize(j // 2, acc2)

    pltpu.sync_copy(out_v, out_hbm.at[pl.ds(wid * ROWS_PER_W, ROWS_PER_W)])


_sc_call = pl.kernel(
    _sc_body,
    out_type=jax.ShapeDtypeStruct((BATCH, 16), jnp.float32),
    mesh=plsc.VectorSubcoreMesh(core_axis_name="c", subcore_axis_name="s"),
    scratch_types=[
        pltpu.VMEM((ROWS_PER_W, HIST), jnp.int32),
        pltpu.VMEM((NBUF, CHMAX, EMBED_DIM), jnp.float32),
        pltpu.VMEM((NUM_CLASSES, EMBED_DIM), jnp.float32),
        pltpu.VMEM((16,), jnp.float32),
        pltpu.VMEM((ROWS_PER_W, 16), jnp.float32),
        pltpu.SemaphoreType.DMA((NBUF,)),
    ],
    compiler_params=pltpu.CompilerParams(
        needs_layout_passes=False, use_tc_tiling_on_sc=False),
)


@jax.jit
def kernel(x, table, W, b):
    wt = W.T.astype(jnp.float32)
    b_pad = jnp.pad(b.astype(jnp.float32), (0, 16 - NUM_CLASSES))
    out16 = _sc_call(table, x.astype(jnp.int32), wt, b_pad)
    return out16[:, :NUM_CLASSES]
